# baseline (speedup 1.0000x reference)
import functools

import jax
import jax.numpy as jnp
from jax import lax
from jax.experimental import pallas as pl
from jax.experimental.pallas import tpu as pltpu
from jax.experimental.pallas import tpu_sc as plsc


def _table_matmul(feat2d, w_t):
    R, C = feat2d.shape
    PD = w_t.shape[1]
    TBLK = 2048
    assert R % TBLK == 0

    def body(f_ref, w_ref, o_ref):
        o_ref[...] = jnp.dot(f_ref[...], w_ref[...],
                             preferred_element_type=jnp.float32)

    return pl.pallas_call(
        body,
        grid=(R // TBLK,),
        in_specs=[pl.BlockSpec((TBLK, C), lambda i: (i, 0)),
                  pl.BlockSpec((C, PD), lambda i: (0, 0))],
        out_specs=pl.BlockSpec((TBLK, PD), lambda i: (i, 0)),
        out_shape=jax.ShapeDtypeStruct((R, PD), jnp.float32),
    )(feat2d, w_t)


def _projection(centers_t, bidx2, P2, R0, Tr, hw, HF, WF):
    NP = centers_t.shape[1]
    BS = P2.shape[0]
    BLK = 2048
    assert NP % BLK == 0
    HWprod = HF * WF

    def body(p2_ref, r0_ref, tr_ref, hw_ref, c_ref, b_ref, idx_ref, wgt_ref,
             m_ref):
        @pl.when(pl.program_id(0) == 0)
        def _():
            for b in range(BS):
                a = [[None] * 4 for _ in range(4)]
                for i in range(4):
                    for j in range(4):
                        s = r0_ref[b, i, 0] * tr_ref[b, 0, j]
                        for k in range(1, 4):
                            s = s + r0_ref[b, i, k] * tr_ref[b, k, j]
                        a[i][j] = s
                for i in range(3):
                    for j in range(4):
                        s = p2_ref[b, i, 0] * a[0][j]
                        for k in range(1, 4):
                            s = s + p2_ref[b, i, k] * a[k][j]
                        m_ref[b, i, j] = s

        c = c_ref[...]
        cx, cy, cz = c[0:1, :], c[1:2, :], c[2:3, :]
        bi = b_ref[...]
        px = jnp.zeros_like(cx)
        py = jnp.zeros_like(cx)
        pz = jnp.zeros_like(cx)
        for b in range(BS):
            sel = bi == b
            pxb = m_ref[b, 0, 0] * cx + m_ref[b, 0, 1] * cy + m_ref[b, 0, 2] * cz + m_ref[b, 0, 3]
            pyb = m_ref[b, 1, 0] * cx + m_ref[b, 1, 1] * cy + m_ref[b, 1, 2] * cz + m_ref[b, 1, 3]
            pzb = m_ref[b, 2, 0] * cx + m_ref[b, 2, 1] * cy + m_ref[b, 2, 2] * cz + m_ref[b, 2, 3]
            px = jnp.where(sel, pxb, px)
            py = jnp.where(sel, pyb, py)
            pz = jnp.where(sel, pzb, pz)
        depth = jnp.maximum(pz, 1e-5)
        u = px / depth
        v = py / depth
        imgw = hw_ref[0, 0]
        imgh = hw_ref[0, 1]
        valid = (pz > 0) & (u >= 0) & (u < imgw) & (v >= 0) & (v < imgh)
        x0 = jnp.floor(u)
        y0 = jnp.floor(v)
        wx1 = u - x0
        wx0 = 1.0 - wx1
        wy1 = v - y0
        wy0 = 1.0 - wy1
        idxs = []
        wgts = []
        for dx, dy, w in ((0, 0, wx0 * wy0), (1, 0, wx1 * wy0),
                          (0, 1, wx0 * wy1), (1, 1, wx1 * wy1)):
            xf = x0 + dx
            yf = y0 + dy
            inb = (xf >= 0) & (xf <= WF - 1) & (yf >= 0) & (yf <= HF - 1)
            xi = jnp.clip(xf, 0, WF - 1).astype(jnp.int32)
            yi = jnp.clip(yf, 0, HF - 1).astype(jnp.int32)
            idxs.append(bi * HWprod + yi * WF + xi)
            wgts.append(jnp.where(valid & inb, w, 0.0))
        idx_ref[...] = jnp.concatenate(idxs, axis=0)
        wgt_ref[...] = jnp.concatenate(wgts, axis=0)

    return pl.pallas_call(
        body,
        grid=(NP // BLK,),
        in_specs=[
            pl.BlockSpec(memory_space=pltpu.SMEM),
            pl.BlockSpec(memory_space=pltpu.SMEM),
            pl.BlockSpec(memory_space=pltpu.SMEM),
            pl.BlockSpec(memory_space=pltpu.SMEM),
            pl.BlockSpec((3, BLK), lambda i: (0, i)),
            pl.BlockSpec((1, BLK), lambda i: (0, i)),
        ],
        out_specs=[
            pl.BlockSpec((4, BLK), lambda i: (0, i)),
            pl.BlockSpec((4, BLK), lambda i: (0, i)),
        ],
        out_shape=[
            jax.ShapeDtypeStruct((4, NP), jnp.int32),
            jax.ShapeDtypeStruct((4, NP), jnp.float32),
        ],
        scratch_shapes=[pltpu.SMEM((BS, 3, 4), jnp.float32)],
    )(P2, R0, Tr, hw, centers_t, bidx2)


def _sc_fuse(table, idx_flat, wgt_flat, point_feat, b_align, NP, CH):
    N, PD = point_feat.shape
    info = plsc.get_sparse_core_info()
    NC, NS = info.num_cores, info.num_subcores
    NW = NC * NS
    PW = NP // NW
    NCHUNK = PW // CH
    assert PW % CH == 0 and CH * 4 == 128
    assert NCHUNK % 2 == 0 and NCHUNK >= 4
    mesh = plsc.VectorSubcoreMesh(core_axis_name="c", subcore_axis_name="s")

    @functools.partial(
        pl.kernel, mesh=mesh,
        out_type=jax.ShapeDtypeStruct((NP, PD), jnp.float32),
        scratch_types=[
            pltpu.VMEM((PW * 4,), jnp.int32),
            pltpu.VMEM((PW * 4,), jnp.float32),
            pltpu.VMEM((CH * 4, PD), jnp.float32),
            pltpu.VMEM((CH * 4, PD), jnp.float32),
            pltpu.VMEM((CH, PD), jnp.float32),
            pltpu.VMEM((CH, PD), jnp.float32),
            pltpu.VMEM((CH, PD), jnp.float32),
            pltpu.VMEM((CH, PD), jnp.float32),
            pltpu.VMEM((PD,), jnp.float32),
            pltpu.SemaphoreType.DMA,
            pltpu.SemaphoreType.DMA,
            pltpu.SemaphoreType.DMA,
            pltpu.SemaphoreType.DMA,
            pltpu.SemaphoreType.DMA,
            pltpu.SemaphoreType.DMA,
        ],
    )
    def fuse(table_h, idx_h, wgt_h, pf_h, bias_h, out_h,
             idx_v, wgt_v, rows0, rows1, pf0, pf1, out0, out1, bias_v,
             gsem0, gsem1, psem0, psem1, osem0, osem1):
        wid = lax.axis_index("s") * NC + lax.axis_index("c")
        base0 = wid * PW
        rows_b = (rows0, rows1)
        pf_b = (pf0, pf1)
        out_b = (out0, out1)
        gsem_b = (gsem0, gsem1)
        psem_b = (psem0, psem1)
        osem_b = (osem0, osem1)

        pltpu.sync_copy(bias_h, bias_v)
        pltpu.sync_copy(idx_h.at[pl.ds(base0 * 4, PW * 4)], idx_v)
        pltpu.sync_copy(wgt_h.at[pl.ds(base0 * 4, PW * 4)], wgt_v)

        def issue(t, b):
            pltpu.async_copy(table_h.at[idx_v.at[pl.ds(t * 128, 128)]],
                             rows_b[b], gsem_b[b])
            pfbase = jnp.minimum(base0 + t * CH, N - CH)
            pltpu.async_copy(pf_h.at[pl.ds(pfbase, CH)], pf_b[b], psem_b[b])

        issue(0, 0)
        issue(1, 1)

        dn = lax.GatherDimensionNumbers(offset_dims=(),
                                        collapsed_slice_dims=(0,),
                                        start_index_map=(0,))

        def step(t, b, bias_regs):
            pltpu.make_async_copy(table_h.at[pl.ds(0, CH * 4)],
                                  rows_b[b], gsem_b[b]).wait()
            pltpu.make_async_copy(pf_h.at[pl.ds(0, CH)],
                                  pf_b[b], psem_b[b]).wait()

            @pl.when(t >= 2)
            def _():
                pltpu.make_async_copy(out_b[b],
                                      out_h.at[pl.ds(base0, CH)],
                                      osem_b[b]).wait()

            def group(g, regs):
                w16 = wgt_v[pl.ds(t * 128 + g * 16, 16)]
                for p in range(4):
                    i = g * 4 + p
                    bi4 = i * 4
                    ws = [lax.gather(w16,
                                     jnp.full((16, 1), 4 * p + c, jnp.int32),
                                     dn, slice_sizes=(1,),
                                     mode=lax.GatherScatterMode.PROMISE_IN_BOUNDS)
                          for c in range(4)]
                    for j in range(PD // 16):
                        acc = pf_b[b][i, pl.ds(16 * j, 16)] + regs[j]
                        for c in range(4):
                            acc = acc + ws[c] * rows_b[b][bi4 + c,
                                                          pl.ds(16 * j, 16)]
                        out_b[b][i, pl.ds(16 * j, 16)] = acc
                return regs

            bias_regs = lax.fori_loop(0, CH // 4, group, bias_regs)
            pltpu.async_copy(out_b[b], out_h.at[pl.ds(base0 + t * CH, CH)],
                             osem_b[b])

            @pl.when(t + 2 < NCHUNK)
            def _():
                issue(t + 2, b)
            return bias_regs

        def pair(tp, bias_regs):
            t0 = tp * 2
            bias_regs = step(t0, 0, bias_regs)
            bias_regs = step(t0 + 1, 1, bias_regs)
            return bias_regs

        bias_regs = tuple(bias_v[pl.ds(16 * j, 16)] for j in range(PD // 16))
        lax.fori_loop(0, NCHUNK // 2, pair, bias_regs)
        pltpu.make_async_copy(out0, out_h.at[pl.ds(base0, CH)], osem0).wait()
        pltpu.make_async_copy(out1, out_h.at[pl.ds(base0, CH)], osem1).wait()

    return fuse(table, idx_flat, wgt_flat, point_feat, b_align)


def kernel(point_feat, pillar_centers, batch_idx, img_feat, P2, R0_rect,
           Tr_velo_to_cam, W_align, b_align, img_h, img_w):
    N, PD = point_feat.shape
    BS, C, HF, WF = img_feat.shape
    info = plsc.get_sparse_core_info()
    NW = info.num_cores * info.num_subcores
    CH = 32
    NP = ((N + NW * CH - 1) // (NW * CH)) * (NW * CH)

    feat2d = img_feat.transpose(0, 2, 3, 1).reshape(BS * HF * WF, C)
    table = _table_matmul(feat2d, W_align.T)

    centers_t = jnp.pad(pillar_centers, ((0, NP - N), (0, 0))).T
    bidx2 = jnp.pad(batch_idx.astype(jnp.int32), (0, NP - N)).reshape(1, NP)
    hw = jnp.stack([jnp.asarray(img_w, jnp.float32),
                    jnp.asarray(img_h, jnp.float32)]).reshape(1, 2)
    idx4, wgt4 = _projection(centers_t, bidx2, P2, R0_rect, Tr_velo_to_cam,
                             hw, HF, WF)
    idx_flat = idx4.T.reshape(NP * 4)
    wgt_flat = wgt4.T.reshape(NP * 4)

    out = _sc_fuse(table, idx_flat, wgt_flat, point_feat, b_align, NP, CH)
    return out[:N]


# speedup vs baseline: 2.5682x; 2.5682x over previous
import functools

import jax
import jax.numpy as jnp
from jax import lax
from jax.experimental import pallas as pl
from jax.experimental.pallas import tpu as pltpu
from jax.experimental.pallas import tpu_sc as plsc


def _table_matmul(feat2d, w_t):
    R, C = feat2d.shape
    PD = w_t.shape[1]
    TBLK = 2048
    assert R % TBLK == 0

    def body(f_ref, w_ref, o_ref):
        o_ref[...] = jnp.dot(f_ref[...], w_ref[...],
                             preferred_element_type=jnp.float32)

    return pl.pallas_call(
        body,
        grid=(R // TBLK,),
        in_specs=[pl.BlockSpec((TBLK, C), lambda i: (i, 0)),
                  pl.BlockSpec((C, PD), lambda i: (0, 0))],
        out_specs=pl.BlockSpec((TBLK, PD), lambda i: (i, 0)),
        out_shape=jax.ShapeDtypeStruct((R, PD), jnp.float32),
    )(feat2d, w_t)


def _projection(centers_t, bidx2, P2, R0, Tr, hw, HF, WF):
    NP = centers_t.shape[1]
    BS = P2.shape[0]
    BLK = 2048
    assert NP % BLK == 0
    HWprod = HF * WF

    def body(p2_ref, r0_ref, tr_ref, hw_ref, c_ref, b_ref, idx_ref, wgt_ref,
             m_ref):
        @pl.when(pl.program_id(0) == 0)
        def _():
            for b in range(BS):
                a = [[None] * 4 for _ in range(4)]
                for i in range(4):
                    for j in range(4):
                        s = r0_ref[b, i, 0] * tr_ref[b, 0, j]
                        for k in range(1, 4):
                            s = s + r0_ref[b, i, k] * tr_ref[b, k, j]
                        a[i][j] = s
                for i in range(3):
                    for j in range(4):
                        s = p2_ref[b, i, 0] * a[0][j]
                        for k in range(1, 4):
                            s = s + p2_ref[b, i, k] * a[k][j]
                        m_ref[b, i, j] = s

        c = c_ref[...]
        cx, cy, cz = c[0:1, :], c[1:2, :], c[2:3, :]
        bi = b_ref[...]
        px = jnp.zeros_like(cx)
        py = jnp.zeros_like(cx)
        pz = jnp.zeros_like(cx)
        for b in range(BS):
            sel = bi == b
            pxb = m_ref[b, 0, 0] * cx + m_ref[b, 0, 1] * cy + m_ref[b, 0, 2] * cz + m_ref[b, 0, 3]
            pyb = m_ref[b, 1, 0] * cx + m_ref[b, 1, 1] * cy + m_ref[b, 1, 2] * cz + m_ref[b, 1, 3]
            pzb = m_ref[b, 2, 0] * cx + m_ref[b, 2, 1] * cy + m_ref[b, 2, 2] * cz + m_ref[b, 2, 3]
            px = jnp.where(sel, pxb, px)
            py = jnp.where(sel, pyb, py)
            pz = jnp.where(sel, pzb, pz)
        depth = jnp.maximum(pz, 1e-5)
        u = px / depth
        v = py / depth
        imgw = hw_ref[0, 0]
        imgh = hw_ref[0, 1]
        valid = (pz > 0) & (u >= 0) & (u < imgw) & (v >= 0) & (v < imgh)
        x0 = jnp.floor(u)
        y0 = jnp.floor(v)
        wx1 = u - x0
        wx0 = 1.0 - wx1
        wy1 = v - y0
        wy0 = 1.0 - wy1
        wgts = []
        for dx, dy, w in ((0, 0, wx0 * wy0), (1, 0, wx1 * wy0),
                          (0, 1, wx0 * wy1), (1, 1, wx1 * wy1)):
            xf = x0 + dx
            yf = y0 + dy
            inb = (xf >= 0) & (xf <= WF - 1) & (yf >= 0) & (yf <= HF - 1)
            wgts.append(jnp.where(valid & inb, w, 0.0))
        xi = jnp.clip(x0, 0, WF - 1).astype(jnp.int32)
        yi = jnp.clip(y0, 0, HF - 1).astype(jnp.int32)
        idx_ref[...] = bi * HWprod + yi * WF + xi
        wgt_ref[...] = jnp.concatenate(wgts, axis=0)

    return pl.pallas_call(
        body,
        grid=(NP // BLK,),
        in_specs=[
            pl.BlockSpec(memory_space=pltpu.SMEM),
            pl.BlockSpec(memory_space=pltpu.SMEM),
            pl.BlockSpec(memory_space=pltpu.SMEM),
            pl.BlockSpec(memory_space=pltpu.SMEM),
            pl.BlockSpec((3, BLK), lambda i: (0, i)),
            pl.BlockSpec((1, BLK), lambda i: (0, i)),
        ],
        out_specs=[
            pl.BlockSpec((1, BLK), lambda i: (0, i)),
            pl.BlockSpec((4, BLK), lambda i: (0, i)),
        ],
        out_shape=[
            jax.ShapeDtypeStruct((1, NP), jnp.int32),
            jax.ShapeDtypeStruct((4, NP), jnp.float32),
        ],
        scratch_shapes=[pltpu.SMEM((BS, 3, 4), jnp.float32)],
    )(P2, R0, Tr, hw, centers_t, bidx2)


def _sc_fuse(table, idx_flat, wgt_flat, point_feat, b_align, NP, CH):
    N, PD = point_feat.shape
    info = plsc.get_sparse_core_info()
    NC, NS = info.num_cores, info.num_subcores
    NW = NC * NS
    PW = NP // NW
    NCHUNK = PW // CH
    assert PW % CH == 0 and CH <= 128 and CH % 8 == 0
    assert NCHUNK % 2 == 0 and NCHUNK >= 4
    mesh = plsc.VectorSubcoreMesh(core_axis_name="c", subcore_axis_name="s")

    @functools.partial(
        pl.kernel, mesh=mesh,
        out_type=jax.ShapeDtypeStruct((NP, PD), jnp.float32),
        scratch_types=[
            pltpu.VMEM((PW,), jnp.int32),
            pltpu.VMEM((PW * 4,), jnp.float32),
            pltpu.VMEM((CH, 4 * PD), jnp.float32),
            pltpu.VMEM((CH, 4 * PD), jnp.float32),
            pltpu.VMEM((CH, PD), jnp.float32),
            pltpu.VMEM((CH, PD), jnp.float32),
            pltpu.VMEM((CH, PD), jnp.float32),
            pltpu.VMEM((CH, PD), jnp.float32),
            pltpu.VMEM((PD,), jnp.float32),
            pltpu.SemaphoreType.DMA,
            pltpu.SemaphoreType.DMA,
            pltpu.SemaphoreType.DMA,
            pltpu.SemaphoreType.DMA,
            pltpu.SemaphoreType.DMA,
            pltpu.SemaphoreType.DMA,
        ],
    )
    def fuse(table_h, idx_h, wgt_h, pf_h, bias_h, out_h,
             idx_v, wgt_v, rows0, rows1, pf0, pf1, out0, out1, bias_v,
             gsem0, gsem1, psem0, psem1, osem0, osem1):
        wid = lax.axis_index("s") * NC + lax.axis_index("c")
        base0 = wid * PW
        rows_b = (rows0, rows1)
        pf_b = (pf0, pf1)
        out_b = (out0, out1)
        gsem_b = (gsem0, gsem1)
        psem_b = (psem0, psem1)
        osem_b = (osem0, osem1)

        pltpu.sync_copy(bias_h, bias_v)
        pltpu.sync_copy(idx_h.at[pl.ds(base0, PW)], idx_v)
        pltpu.sync_copy(wgt_h.at[pl.ds(base0 * 4, PW * 4)], wgt_v)

        def issue(t, b):
            pltpu.async_copy(table_h.at[idx_v.at[pl.ds(t * CH, CH)]],
                             rows_b[b], gsem_b[b])
            pfbase = jnp.minimum(base0 + t * CH, N - CH)
            pltpu.async_copy(pf_h.at[pl.ds(pfbase, CH)], pf_b[b], psem_b[b])

        issue(0, 0)
        issue(1, 1)

        dn = lax.GatherDimensionNumbers(offset_dims=(),
                                        collapsed_slice_dims=(0,),
                                        start_index_map=(0,))

        def step(t, b, bias_regs):
            pltpu.make_async_copy(table_h.at[pl.ds(0, CH)],
                                  rows_b[b], gsem_b[b]).wait()
            pltpu.make_async_copy(pf_h.at[pl.ds(0, CH)],
                                  pf_b[b], psem_b[b]).wait()

            @pl.when(t >= 2)
            def _():
                pltpu.make_async_copy(out_b[b],
                                      out_h.at[pl.ds(base0, CH)],
                                      osem_b[b]).wait()

            def group(g, regs):
                w16 = wgt_v[pl.ds(t * CH * 4 + g * 16, 16)]
                for p in range(4):
                    i = g * 4 + p
                    ws = [lax.gather(w16,
                                     jnp.full((16, 1), 4 * p + c, jnp.int32),
                                     dn, slice_sizes=(1,),
                                     mode=lax.GatherScatterMode.PROMISE_IN_BOUNDS)
                          for c in range(4)]
                    for j in range(PD // 16):
                        acc = pf_b[b][i, pl.ds(16 * j, 16)] + regs[j]
                        for c in range(4):
                            acc = acc + ws[c] * rows_b[b][i,
                                                          pl.ds(128 * c + 16 * j, 16)]
                        out_b[b][i, pl.ds(16 * j, 16)] = acc
                return regs

            bias_regs = lax.fori_loop(0, CH // 4, group, bias_regs)
            pltpu.async_copy(out_b[b], out_h.at[pl.ds(base0 + t * CH, CH)],
                             osem_b[b])

            @pl.when(t + 2 < NCHUNK)
            def _():
                issue(t + 2, b)
            return bias_regs

        def pair(tp, bias_regs):
            t0 = tp * 2
            bias_regs = step(t0, 0, bias_regs)
            bias_regs = step(t0 + 1, 1, bias_regs)
            return bias_regs

        bias_regs = tuple(bias_v[pl.ds(16 * j, 16)] for j in range(PD // 16))
        lax.fori_loop(0, NCHUNK // 2, pair, bias_regs)
        pltpu.make_async_copy(out0, out_h.at[pl.ds(base0, CH)], osem0).wait()
        pltpu.make_async_copy(out1, out_h.at[pl.ds(base0, CH)], osem1).wait()

    return fuse(table, idx_flat, wgt_flat, point_feat, b_align)


def kernel(point_feat, pillar_centers, batch_idx, img_feat, P2, R0_rect,
           Tr_velo_to_cam, W_align, b_align, img_h, img_w):
    N, PD = point_feat.shape
    BS, C, HF, WF = img_feat.shape
    info = plsc.get_sparse_core_info()
    NW = info.num_cores * info.num_subcores
    CH = 32
    NP = ((N + NW * CH - 1) // (NW * CH)) * (NW * CH)

    feat2d = img_feat.transpose(0, 2, 3, 1).reshape(BS * HF * WF, C)
    table = _table_matmul(feat2d, W_align.T)
    table = jnp.concatenate(
        [table, jnp.roll(table, -1, 0), jnp.roll(table, -WF, 0),
         jnp.roll(table, -(WF + 1), 0)], axis=1)

    centers_t = jnp.pad(pillar_centers, ((0, NP - N), (0, 0))).T
    bidx2 = jnp.pad(batch_idx.astype(jnp.int32), (0, NP - N)).reshape(1, NP)
    hw = jnp.stack([jnp.asarray(img_w, jnp.float32),
                    jnp.asarray(img_h, jnp.float32)]).reshape(1, 2)
    idx1, wgt4 = _projection(centers_t, bidx2, P2, R0_rect, Tr_velo_to_cam,
                             hw, HF, WF)
    idx_flat = idx1.reshape(NP)
    wgt_flat = wgt4.T.reshape(NP * 4)

    out = _sc_fuse(table, idx_flat, wgt_flat, point_feat, b_align, NP, CH)
    return out[:N]
